# baseline (speedup 1.0000x reference)
import jax
import jax.numpy as jnp
from jax.experimental import pallas as pl
from jax.experimental.pallas import tpu as pltpu

BM = 512
ETILE = 8


def _cast_kernel(w_ref, wbf_ref):
    wbf_ref[:] = w_ref[:].astype(jnp.bfloat16)


def _router_kernel(x_ref, w_ref, b_ref, gating_ref, idx_ref, logits_ref):
    logits_ref[:] = jnp.dot(x_ref[:].astype(jnp.bfloat16), w_ref[:],
                            preferred_element_type=jnp.float32)
    b = b_ref[:]
    D = b.shape[-1]
    iota = jax.lax.broadcasted_iota(jnp.int32, (ETILE, D), 1)

    def tile(i, _):
        rows = pl.ds(i * ETILE, ETILE)
        t = logits_ref[rows, :] + b
        row_max = jnp.max(t, axis=-1, keepdims=True)
        e = jnp.exp(t - row_max)
        denom = jnp.sum(e, axis=-1, keepdims=True)
        gating_ref[rows, :] = e / denom
        cand = jnp.where(t == row_max, iota, jnp.int32(2**30))
        idx_ref[rows, :] = jnp.min(cand, axis=-1, keepdims=True)
        return 0

    jax.lax.fori_loop(0, BM // ETILE, tile, 0, unroll=2)


def kernel(x, gate_W, gate_b):
    B, S, D = x.shape
    M = B * S
    x2 = x.reshape(M, D)
    b2 = gate_b.reshape(1, D)

    w_bf16 = pl.pallas_call(
        _cast_kernel,
        grid=(8,),
        in_specs=[pl.BlockSpec((D // 8, D), lambda i: (i, 0))],
        out_specs=pl.BlockSpec((D // 8, D), lambda i: (i, 0)),
        out_shape=jax.ShapeDtypeStruct((D, D), jnp.bfloat16),
    )(gate_W)

    gating, idx = pl.pallas_call(
        _router_kernel,
        grid=(M // BM,),
        in_specs=[
            pl.BlockSpec((BM, D), lambda i: (i, 0)),
            pl.BlockSpec((D, D), lambda i: (0, 0)),
            pl.BlockSpec((1, D), lambda i: (0, 0)),
        ],
        out_specs=[
            pl.BlockSpec((BM, D), lambda i: (i, 0)),
            pl.BlockSpec((BM, 1), lambda i: (i, 0)),
        ],
        out_shape=[
            jax.ShapeDtypeStruct((M, D), jnp.float32),
            jax.ShapeDtypeStruct((M, 1), jnp.int32),
        ],
        scratch_shapes=[pltpu.VMEM((BM, D), jnp.float32)],
        compiler_params=pltpu.CompilerParams(
            dimension_semantics=("arbitrary",),
        ),
    )(x2, w_bf16, b2)
    return gating.reshape(B, S, D), idx.reshape(B, S)


# speedup vs baseline: 1.9285x; 1.9285x over previous
import jax
import jax.numpy as jnp
from jax.experimental import pallas as pl
from jax.experimental.pallas import tpu as pltpu

BM = 512
ETILE = 8


def _cast_kernel(w_ref, wbf_ref):
    wbf_ref[:] = w_ref[:].astype(jnp.bfloat16)


def _router_kernel(x_ref, w_ref, b_ref, gating_ref, idx_ref, logits_ref):
    del logits_ref
    logits = jnp.dot(x_ref[:].astype(jnp.bfloat16), w_ref[:],
                     preferred_element_type=jnp.float32) + b_ref[:]
    row_max = jnp.max(logits, axis=-1, keepdims=True)
    e = jnp.exp(logits - row_max)
    denom = jnp.sum(e, axis=-1, keepdims=True)
    gating_ref[:] = e / denom
    iota = jax.lax.broadcasted_iota(jnp.int32, logits.shape, 1)
    cand = jnp.where(logits == row_max, iota, jnp.int32(2**30))
    idx_ref[:] = jnp.min(cand, axis=-1, keepdims=True)


def kernel(x, gate_W, gate_b):
    B, S, D = x.shape
    M = B * S
    x2 = x.reshape(M, D)
    b2 = gate_b.reshape(1, D)

    w_bf16 = pl.pallas_call(
        _cast_kernel,
        grid=(8,),
        in_specs=[pl.BlockSpec((D // 8, D), lambda i: (i, 0))],
        out_specs=pl.BlockSpec((D // 8, D), lambda i: (i, 0)),
        out_shape=jax.ShapeDtypeStruct((D, D), jnp.bfloat16),
    )(gate_W)

    gating, idx = pl.pallas_call(
        _router_kernel,
        grid=(M // BM,),
        in_specs=[
            pl.BlockSpec((BM, D), lambda i: (i, 0)),
            pl.BlockSpec((D, D), lambda i: (0, 0)),
            pl.BlockSpec((1, D), lambda i: (0, 0)),
        ],
        out_specs=[
            pl.BlockSpec((BM, D), lambda i: (i, 0)),
            pl.BlockSpec((BM, 1), lambda i: (i, 0)),
        ],
        out_shape=[
            jax.ShapeDtypeStruct((M, D), jnp.float32),
            jax.ShapeDtypeStruct((M, 1), jnp.int32),
        ],
        scratch_shapes=[pltpu.VMEM((BM, D), jnp.float32)],
        compiler_params=pltpu.CompilerParams(
            dimension_semantics=("arbitrary",),
        ),
    )(x2, w_bf16, b2)
    return gating.reshape(B, S, D), idx.reshape(B, S)
